# baseline (speedup 1.0000x reference)
import functools

import jax
import jax.numpy as jnp
from jax import lax
from jax.experimental import pallas as pl
from jax.experimental.pallas import tpu as pltpu
from jax.experimental.pallas import tpu_sc as plsc

_TEMP = 0.07
_B, _C, _H, _W = 8, 256, 32, 32
_P = 32
_HW = _H * _W
_Q = 4
_QP = _HW // _Q
_LANES = 16


def _sc_route_body(lab_hbm, oh_hbm, lab_v, oh_v, sem):
    wid = lax.axis_index("s") * 2 + lax.axis_index("c")
    b = wid // _Q
    q = wid % _Q
    lane = jnp.arange(_LANES, dtype=jnp.int32)

    cp_lab = pltpu.async_copy(lab_hbm.at[b], lab_v, sem)

    zf = jnp.zeros((_LANES,), jnp.float32)

    def _zero(m, _):
        for j in range(_QP // _LANES):
            oh_v[m, pl.ds(j * _LANES, _LANES)] = zf
        return 0

    lax.fori_loop(0, _P, _zero, 0)
    cp_lab.wait()

    ones = jnp.ones((_LANES,), jnp.float32)
    for v in range(_QP // _LANES):
        p0 = q * _QP + v * _LANES
        cls = lab_v[p0 // 128, pl.ds(p0 % 128, _LANES)]
        plsc.store_scatter(oh_v, [cls, lane + v * _LANES], ones)

    pltpu.sync_copy(oh_v, oh_hbm.at[b, :, pl.ds(q * _QP, _QP)])


_sc_route = pl.kernel(
    _sc_route_body,
    mesh=plsc.VectorSubcoreMesh(core_axis_name="c", subcore_axis_name="s"),
    compiler_params=pltpu.CompilerParams(use_tc_tiling_on_sc=True,
                                         needs_layout_passes=False,
                                         disable_bounds_checks=True,
                                         skip_device_barrier=True),
    out_type=[
        jax.ShapeDtypeStruct((_B, _P, _HW), jnp.float32),
    ],
    scratch_types=[
        pltpu.VMEM((8, 128), jnp.int32),
        pltpu.VMEM((_P, _QP), jnp.float32),
        pltpu.SemaphoreType.DMA,
    ],
)


def _loss_body(nc_ref, oh_ref, feat_ref, proto_ref, out_ref):
    oh = oh_ref[...]
    feat = feat_ref[...]

    sums = lax.dot_general(oh, feat, (((2,), (2,)), ((0,), (0,))),
                           preferred_element_type=jnp.float32)
    cnt2 = jnp.sum(oh, axis=2)

    nc = nc_ref[0]
    cls1 = lax.broadcasted_iota(jnp.int32, (1, _P), 1)
    valid2 = (cls1 >= 1) & (cls1 <= nc)
    pres2 = (cnt2 > 0.5) & valid2
    presf2 = pres2.astype(jnp.float32)
    np_ = jnp.sum(presf2, axis=0)
    nrm = jnp.sqrt(jnp.sum(sums * sums, axis=2, keepdims=True))
    u = sums / jnp.maximum(nrm, 1e-12)
    pr = proto_ref[...]
    pnrm = jnp.sqrt(jnp.sum(pr * pr, axis=1, keepdims=True))
    pn = pr / jnp.maximum(pnrm, 1e-12)

    a_mat = u.reshape(_B * _P, _C)
    g1 = lax.dot_general(a_mat, a_mat, (((1,), (1,)), ((), ())),
                         preferred_element_type=jnp.float32)
    g2 = lax.dot_general(a_mat, pn, (((1,), (1,)), ((), ())),
                         preferred_element_type=jnp.float32)
    e1 = jnp.exp(g1 * (1.0 / _TEMP)).reshape(_B, _P, _B * _P)
    e2 = jnp.exp(g2 * (1.0 / _TEMP)).reshape(_B, _P, _P)

    inv_cnt = 1.0 / (np_ + 1.0)
    w12d = presf2 * inv_cnt[None, :]
    w1f = jnp.concatenate([w12d[i:i + 1, :] for i in range(_B)],
                          axis=1)
    w2 = valid2[0].astype(jnp.float32) * inv_cnt
    den = (jnp.sum(e1 * w1f[None, :, :], axis=2)
           + jnp.sum(e2 * w2[None, None, :], axis=2))

    q = jnp.sum(u, axis=0)
    nm1 = jnp.sum(u * q[None, :, :], axis=2)
    nm2 = jnp.sum(u * pn[None, :, :], axis=2)

    t = np_[None, :] * jnp.log(den) - (nm1 + nm2 - 1.0) * (1.0 / _TEMP)
    cls_sum = jnp.sum(t * presf2, axis=0)
    contrib = cls_sum / jnp.maximum(np_ * np_, 1.0)
    exist = (np_ >= 0.5).astype(jnp.float32)
    loss = 0.1 * jnp.sum(contrib * exist) / jnp.sum(exist)
    out_ref[0, 0] = loss


@jax.jit
def _run(labels, features, prototypes, num_class):
    lab_rows = lax.optimization_barrier(labels[:, ::16, :])
    lab_ds = lab_rows[:, :, ::16].astype(jnp.int32).reshape(_B, 8, 128)
    feat = features.reshape(_B, _C, _HW)
    proto_p = jnp.zeros((_P, _C), jnp.float32).at[:21].set(
        prototypes.astype(jnp.float32))
    nc_arr = jnp.asarray(num_class, jnp.int32).reshape(1)

    (oh,) = _sc_route(lab_ds)

    out = pl.pallas_call(
        _loss_body,
        in_specs=[
            pl.BlockSpec(memory_space=pltpu.SMEM),
            pl.BlockSpec((_B, _P, _HW), lambda: (0, 0, 0)),
            pl.BlockSpec((_B, _C, _HW), lambda: (0, 0, 0)),
            pl.BlockSpec((_P, _C), lambda: (0, 0)),
        ],
        out_specs=pl.BlockSpec(memory_space=pltpu.SMEM),
        out_shape=jax.ShapeDtypeStruct((1, 1), jnp.float32),
    )(nc_arr, oh, feat, proto_p)
    return out[0, 0]


def kernel(labels, features_old, features, outputs_old, outputs, prototypes,
           num_class, num_old_class, num_new_class, epoch, train_step,
           len_epoch):
    return _run(labels, features, prototypes, num_class)


# speedup vs baseline: 1.0841x; 1.0841x over previous
import functools

import jax
import jax.numpy as jnp
from jax import lax
from jax.experimental import pallas as pl
from jax.experimental.pallas import tpu as pltpu
from jax.experimental.pallas import tpu_sc as plsc

_TEMP = 0.07
_B, _C, _H, _W = 8, 256, 32, 32
_P = 32
_HW = _H * _W
_Q = 4
_QP = _HW // _Q
_LANES = 16


def _sc_route_body(lab_hbm, oh_hbm, idx_v, rows_v, oh_v, sem):
    wid = lax.axis_index("s") * 2 + lax.axis_index("c")
    b = wid // _Q
    q = wid % _Q
    lane = jnp.arange(_LANES, dtype=jnp.int32)

    idx_v[pl.ds(0, _LANES)] = b * 512 + lane * 16
    idx_v[pl.ds(_LANES, _LANES)] = b * 512 + (lane + _LANES) * 16
    cp_lab = pltpu.async_copy(lab_hbm.at[idx_v], rows_v, sem)

    zf = jnp.zeros((_LANES,), jnp.float32)

    def _zero(m, _):
        for j in range(_QP // _LANES):
            oh_v[m, pl.ds(j * _LANES, _LANES)] = zf
        return 0

    lax.fori_loop(0, _P, _zero, 0)
    cp_lab.wait()

    ones = jnp.ones((_LANES,), jnp.float32)
    mask0 = lane == 0
    for r in range(_H // _Q):
        row = q * (_H // _Q) + r
        for k in range(_W):
            cls = rows_v[row, pl.ds(k * _LANES, _LANES)]
            col = jnp.full((_LANES,), r * _W + k, jnp.int32)
            plsc.store_scatter(oh_v, [cls, col], ones, mask=mask0)

    pltpu.sync_copy(oh_v, oh_hbm.at[b, :, pl.ds(q * _QP, _QP)])


_sc_route = pl.kernel(
    _sc_route_body,
    mesh=plsc.VectorSubcoreMesh(core_axis_name="c", subcore_axis_name="s"),
    compiler_params=pltpu.CompilerParams(use_tc_tiling_on_sc=True,
                                         needs_layout_passes=False),
    out_type=[
        jax.ShapeDtypeStruct((_B, _P, _HW), jnp.float32),
    ],
    scratch_types=[
        pltpu.VMEM((2 * _LANES,), jnp.int32),
        pltpu.VMEM((_H, 512), jnp.int32),
        pltpu.VMEM((_P, _QP), jnp.float32),
        pltpu.SemaphoreType.DMA,
    ],
)


def _loss_body(nc_ref, oh_ref, feat_ref, proto_ref, out_ref):
    oh = oh_ref[...]
    feat = feat_ref[...]

    sums = lax.dot_general(oh, feat, (((2,), (2,)), ((0,), (0,))),
                           preferred_element_type=jnp.float32)
    cnt2 = jnp.sum(oh, axis=2)

    nc = nc_ref[0]
    cls1 = lax.broadcasted_iota(jnp.int32, (1, _P), 1)
    valid2 = (cls1 >= 1) & (cls1 <= nc)
    pres2 = (cnt2 > 0.5) & valid2
    presf2 = pres2.astype(jnp.float32)
    np_ = jnp.sum(presf2, axis=0)
    nrm = jnp.sqrt(jnp.sum(sums * sums, axis=2, keepdims=True))
    u = sums / jnp.maximum(nrm, 1e-12)
    pr = proto_ref[...]
    pnrm = jnp.sqrt(jnp.sum(pr * pr, axis=1, keepdims=True))
    pn = pr / jnp.maximum(pnrm, 1e-12)

    a_mat = u.reshape(_B * _P, _C)
    g1 = lax.dot_general(a_mat, a_mat, (((1,), (1,)), ((), ())),
                         preferred_element_type=jnp.float32)
    g2 = lax.dot_general(a_mat, pn, (((1,), (1,)), ((), ())),
                         preferred_element_type=jnp.float32)
    e1 = jnp.exp(g1 * (1.0 / _TEMP)).reshape(_B, _P, _B * _P)
    e2 = jnp.exp(g2 * (1.0 / _TEMP)).reshape(_B, _P, _P)

    inv_cnt = 1.0 / (np_ + 1.0)
    w12d = presf2 * inv_cnt[None, :]
    w1f = jnp.concatenate([w12d[i:i + 1, :] for i in range(_B)],
                          axis=1)
    w2 = valid2[0].astype(jnp.float32) * inv_cnt
    den = (jnp.sum(e1 * w1f[None, :, :], axis=2)
           + jnp.sum(e2 * w2[None, None, :], axis=2))

    q = jnp.sum(u, axis=0)
    nm1 = jnp.sum(u * q[None, :, :], axis=2)
    nm2 = jnp.sum(u * pn[None, :, :], axis=2)

    t = np_[None, :] * jnp.log(den) - (nm1 + nm2 - 1.0) * (1.0 / _TEMP)
    cls_sum = jnp.sum(t * presf2, axis=0)
    contrib = cls_sum / jnp.maximum(np_ * np_, 1.0)
    exist = (np_ >= 0.5).astype(jnp.float32)
    loss = 0.1 * jnp.sum(contrib * exist) / jnp.sum(exist)
    out_ref[0, 0] = loss


@jax.jit
def _run(labels, features, prototypes, num_class):
    lab2d = labels.astype(jnp.int32).reshape(_B * 512, 512)
    feat = features.reshape(_B, _C, _HW)
    proto_p = jnp.zeros((_P, _C), jnp.float32).at[:21].set(
        prototypes.astype(jnp.float32))
    nc_arr = jnp.asarray(num_class, jnp.int32).reshape(1)

    (oh,) = _sc_route(lab2d)

    out = pl.pallas_call(
        _loss_body,
        in_specs=[
            pl.BlockSpec(memory_space=pltpu.SMEM),
            pl.BlockSpec((_B, _P, _HW), lambda: (0, 0, 0)),
            pl.BlockSpec((_B, _C, _HW), lambda: (0, 0, 0)),
            pl.BlockSpec((_P, _C), lambda: (0, 0)),
        ],
        out_specs=pl.BlockSpec(memory_space=pltpu.SMEM),
        out_shape=jax.ShapeDtypeStruct((1, 1), jnp.float32),
    )(nc_arr, oh, feat, proto_p)
    return out[0, 0]


def kernel(labels, features_old, features, outputs_old, outputs, prototypes,
           num_class, num_old_class, num_new_class, epoch, train_step,
           len_epoch):
    return _run(labels, features, prototypes, num_class)


# speedup vs baseline: 1.0846x; 1.0005x over previous
import jax
import jax.numpy as jnp
from jax import lax
from jax.experimental import pallas as pl
from jax.experimental.pallas import tpu as pltpu
from jax.experimental.pallas import tpu_sc as plsc

_TEMP = 0.07
_B, _C, _H, _W = 8, 256, 32, 32
_P = 32
_HW = _H * _W
_Q = 4
_QP = _HW // _Q
_LANES = 16


def _sc_route_body(lab_hbm, oh_hbm, idx_v, rows_v, oh_v, sem):
    wid = lax.axis_index("s") * 2 + lax.axis_index("c")
    b = wid // _Q
    q = wid % _Q
    lane = jnp.arange(_LANES, dtype=jnp.int32)

    idx_v[pl.ds(0, _LANES)] = b * 512 + lane * 16
    idx_v[pl.ds(_LANES, _LANES)] = b * 512 + (lane + _LANES) * 16
    cp_lab = pltpu.async_copy(lab_hbm.at[idx_v], rows_v, sem)

    zf = jnp.zeros((_LANES,), jnp.float32)

    def _zero(m, _):
        for j in range(_QP // _LANES):
            oh_v[m, pl.ds(j * _LANES, _LANES)] = zf
        return 0

    lax.fori_loop(0, _P, _zero, 0)
    cp_lab.wait()

    ones = jnp.ones((_LANES,), jnp.float32)
    mask0 = lane == 0
    for r in range(_H // _Q):
        row = q * (_H // _Q) + r
        for k in range(_W):
            cls = rows_v[row, pl.ds(k * _LANES, _LANES)]
            col = jnp.full((_LANES,), r * _W + k, jnp.int32)
            plsc.store_scatter(oh_v, [cls, col], ones, mask=mask0)

    pltpu.sync_copy(oh_v, oh_hbm.at[b, :, pl.ds(q * _QP, _QP)])


_sc_route = pl.kernel(
    _sc_route_body,
    mesh=plsc.VectorSubcoreMesh(core_axis_name="c", subcore_axis_name="s"),
    compiler_params=pltpu.CompilerParams(use_tc_tiling_on_sc=True,
                                         needs_layout_passes=False),
    out_type=[
        jax.ShapeDtypeStruct((_B, _P, _HW), jnp.float32),
    ],
    scratch_types=[
        pltpu.VMEM((2 * _LANES,), jnp.int32),
        pltpu.VMEM((_H, 512), jnp.int32),
        pltpu.VMEM((_P, _QP), jnp.float32),
        pltpu.SemaphoreType.DMA,
    ],
)


def _loss_body(nc_ref, oh_ref, feat_ref, proto_ref, out_ref):
    oh = oh_ref[...]
    feat = feat_ref[...]

    sums = lax.dot_general(oh, feat, (((2,), (2,)), ((0,), (0,))),
                           preferred_element_type=jnp.float32)
    cnt2 = jnp.sum(oh, axis=2)

    nc = nc_ref[0]
    cls1 = lax.broadcasted_iota(jnp.int32, (1, _P), 1)
    valid2 = (cls1 >= 1) & (cls1 <= nc)
    pres2 = (cnt2 > 0.5) & valid2
    presf2 = pres2.astype(jnp.float32)
    np_ = jnp.sum(presf2, axis=0)
    nrm = jnp.sqrt(jnp.sum(sums * sums, axis=2, keepdims=True))
    u = sums / jnp.maximum(nrm, 1e-12)
    pr = proto_ref[...]
    pnrm = jnp.sqrt(jnp.sum(pr * pr, axis=1, keepdims=True))
    pn = pr / jnp.maximum(pnrm, 1e-12)

    a_mat = u.reshape(_B * _P, _C)
    g1 = lax.dot_general(a_mat, a_mat, (((1,), (1,)), ((), ())),
                         preferred_element_type=jnp.float32)
    g2 = lax.dot_general(a_mat, pn, (((1,), (1,)), ((), ())),
                         preferred_element_type=jnp.float32)
    e1 = jnp.exp(g1 * (1.0 / _TEMP)).reshape(_B, _P, _B * _P)
    e2 = jnp.exp(g2 * (1.0 / _TEMP)).reshape(_B, _P, _P)

    inv_cnt = 1.0 / (np_ + 1.0)
    w12d = presf2 * inv_cnt[None, :]
    w1f = jnp.concatenate([w12d[i:i + 1, :] for i in range(_B)],
                          axis=1)
    w2 = valid2[0].astype(jnp.float32) * inv_cnt
    den = (jnp.sum(e1 * w1f[None, :, :], axis=2)
           + jnp.sum(e2 * w2[None, None, :], axis=2))

    q = jnp.sum(u, axis=0)
    nm1 = jnp.sum(u * q[None, :, :], axis=2)
    nm2 = jnp.sum(u * pn[None, :, :], axis=2)

    t = np_[None, :] * jnp.log(den) - (nm1 + nm2 - 1.0) * (1.0 / _TEMP)
    cls_sum = jnp.sum(t * presf2, axis=0)
    contrib = cls_sum / jnp.maximum(np_ * np_, 1.0)
    exist = (np_ >= 0.5).astype(jnp.float32)
    loss = 0.1 * jnp.sum(contrib * exist) / jnp.sum(exist)
    out_ref[0, 0] = loss


@jax.jit
def _run(labels, features, prototypes, num_class):
    lab2d = labels.astype(jnp.int32).reshape(_B * 512, 512)
    feat = features.reshape(_B, _C, _HW)
    proto_p = jnp.zeros((_P, _C), jnp.float32).at[:21].set(
        prototypes.astype(jnp.float32))
    nc_arr = jnp.asarray(num_class, jnp.int32).reshape(1)

    (oh,) = _sc_route(lab2d)

    out = pl.pallas_call(
        _loss_body,
        in_specs=[
            pl.BlockSpec(memory_space=pltpu.SMEM),
            pl.BlockSpec((_B, _P, _HW), lambda: (0, 0, 0)),
            pl.BlockSpec((_B, _C, _HW), lambda: (0, 0, 0)),
            pl.BlockSpec((_P, _C), lambda: (0, 0)),
        ],
        out_specs=pl.BlockSpec(memory_space=pltpu.SMEM),
        out_shape=jax.ShapeDtypeStruct((1, 1), jnp.float32),
    )(nc_arr, oh, feat, proto_p)
    return out[0, 0]


def kernel(labels, features_old, features, outputs_old, outputs, prototypes,
           num_class, num_old_class, num_new_class, epoch, train_step,
           len_epoch):
    return _run(labels, features, prototypes, num_class)
